# baseline (speedup 1.0000x reference)
import functools

import jax
import jax.numpy as jnp
from jax import lax
from jax.experimental import pallas as pl
from jax.experimental.pallas import tpu as pltpu
from jax.experimental.pallas import tpu_sc as plsc

BATCH = 16384
EMB_DIM = 32
NUM_WORKERS = 32
B_PER_W = BATCH // NUM_WORKERS
IDX_CHUNK = 128
N_CHUNKS = B_PER_W // IDX_CHUNK
GROUPS = B_PER_W // 16


def _ncf_body(uid_hbm, iid_hbm, utab_hbm, itab_hbm, wb_hbm, out_hbm,
              uidx_v, iidx_v, urows_v, irows_v, wb_v, out_v, sem):
    cid = lax.axis_index("c")
    sid = lax.axis_index("s")
    wid = sid * 2 + cid
    base = wid * B_PER_W

    for c in range(N_CHUNKS):
        pltpu.sync_copy(uid_hbm.at[pl.ds(base + c * IDX_CHUNK, IDX_CHUNK)],
                        uidx_v.at[c])
        pltpu.sync_copy(iid_hbm.at[pl.ds(base + c * IDX_CHUNK, IDX_CHUNK)],
                        iidx_v.at[c])
    pltpu.sync_copy(wb_hbm, wb_v)

    copies = []
    for c in range(N_CHUNKS):
        copies.append(pltpu.async_copy(
            utab_hbm.at[uidx_v.at[c]],
            urows_v.at[pl.ds(c * IDX_CHUNK, IDX_CHUNK)], sem))
        copies.append(pltpu.async_copy(
            itab_hbm.at[iidx_v.at[c]],
            irows_v.at[pl.ds(c * IDX_CHUNK, IDX_CHUNK)], sem))
    for cp in copies:
        cp.wait()

    lane = lax.iota(jnp.int32, 16)
    bias = plsc.load_gather(
        wb_v, [jnp.full((16,), 2 * EMB_DIM + 1, jnp.int32)])

    def group_body(g, _):
        row_idx = g * 16 + lane
        acc = bias
        for j in range(EMB_DIM):
            col = jnp.full((16,), j, jnp.int32)
            wu = plsc.load_gather(wb_v, [jnp.full((16,), j + 1, jnp.int32)])
            wi = plsc.load_gather(
                wb_v, [jnp.full((16,), EMB_DIM + j + 1, jnp.int32)])
            uv = plsc.load_gather(urows_v, [row_idx, col])
            iv = plsc.load_gather(irows_v, [row_idx, col])
            acc = acc + uv * wu + iv * wi
        out_v[pl.ds(g * 16, 16)] = acc
        return 0

    lax.fori_loop(0, GROUPS, group_body, 0)

    pltpu.sync_copy(out_v, out_hbm.at[pl.ds(base, B_PER_W)])


@jax.jit
def _ncf(user_ids, item_ids, user_table, item_table, wb):
    mesh = plsc.VectorSubcoreMesh(core_axis_name="c", subcore_axis_name="s")
    kern = functools.partial(
        pl.kernel,
        mesh=mesh,
        compiler_params=pltpu.CompilerParams(
            needs_layout_passes=False, use_tc_tiling_on_sc=False),
        out_type=jax.ShapeDtypeStruct((BATCH,), jnp.float32),
        scratch_types=[
            pltpu.VMEM((N_CHUNKS, IDX_CHUNK), jnp.int32),
            pltpu.VMEM((N_CHUNKS, IDX_CHUNK), jnp.int32),
            pltpu.VMEM((B_PER_W, EMB_DIM), jnp.float32),
            pltpu.VMEM((B_PER_W, EMB_DIM), jnp.float32),
            pltpu.VMEM((80,), jnp.float32),
            pltpu.VMEM((B_PER_W,), jnp.float32),
            pltpu.SemaphoreType.DMA,
        ],
    )(_ncf_body)
    return kern(user_ids, item_ids, user_table, item_table, wb)


def kernel(user_ids, item_ids, user_table, item_table, W, b):
    wb = jnp.concatenate(
        [jnp.zeros((1,), jnp.float32), W.reshape(2 * EMB_DIM), b,
         jnp.zeros((14,), jnp.float32)])
    out = _ncf(user_ids.astype(jnp.int32), item_ids.astype(jnp.int32),
               user_table, item_table, wb)
    return out.reshape(BATCH, 1)


# speedup vs baseline: 1.5408x; 1.5408x over previous
import functools

import jax
import jax.numpy as jnp
from jax import lax
from jax.experimental import pallas as pl
from jax.experimental.pallas import tpu as pltpu
from jax.experimental.pallas import tpu_sc as plsc

BATCH = 16384
EMB_DIM = 32
NUM_WORKERS = 32
B_PER_W = BATCH // NUM_WORKERS
GROUPS = B_PER_W // 16
N_USERS = 1000000
N_ITEMS = 100000


def _ncf_body(uid_hbm, iid_hbm, utab_hbm, itab_hbm, wb_hbm, out_hbm,
              idx_v, rows_v, wb_v, out_v, sem):
    cid = lax.axis_index("c")
    sid = lax.axis_index("s")
    wid = sid * 2 + cid
    base = wid * B_PER_W

    pltpu.sync_copy(wb_hbm, wb_v)
    lane = lax.iota(jnp.int32, 16)
    bias = plsc.load_gather(
        wb_v, [jnp.full((16,), 2 * EMB_DIM + 1, jnp.int32)])

    utab3 = utab_hbm.reshape(N_USERS // 8, 8, EMB_DIM)
    itab3 = itab_hbm.reshape(N_ITEMS // 8, 8, EMB_DIM)

    def run_pass(id_hbm, tab3, w_base, first):
        pltpu.sync_copy(id_hbm.at[pl.ds(base, B_PER_W)], idx_v)

        def dma_body(k, _):
            v = idx_v[pl.ds(k * 16, 16)]
            slab = lax.shift_right_logical(v, 3)
            for l in range(16):
                pltpu.async_copy(
                    tab3.at[slab[l], pl.ds(v[l] & 7, 1)],
                    rows_v.at[2 * k + l // 8, pl.ds(l % 8, 1)], sem)
            return 0

        lax.fori_loop(0, GROUPS, dma_body, 0)
        pltpu.make_async_copy(
            tab3.at[pl.ds(0, B_PER_W // 8)], rows_v, sem).wait()

        def group_body(g, _):
            n = g * 16 + lane
            slab = lax.shift_right_logical(n, 3)
            r = n & 7
            if first:
                acc = bias
            else:
                acc = out_v[pl.ds(g * 16, 16)]
            for j in range(EMB_DIM):
                w = plsc.load_gather(
                    wb_v, [jnp.full((16,), w_base + j + 1, jnp.int32)])
                vals = plsc.load_gather(
                    rows_v, [slab, r, jnp.full((16,), j, jnp.int32)])
                acc = acc + vals * w
            out_v[pl.ds(g * 16, 16)] = acc
            return 0

        lax.fori_loop(0, GROUPS, group_body, 0)

    run_pass(uid_hbm, utab3, 0, True)
    run_pass(iid_hbm, itab3, EMB_DIM, False)

    pltpu.sync_copy(out_v, out_hbm.at[pl.ds(base, B_PER_W)])


@jax.jit
def _ncf(user_ids, item_ids, user_table, item_table, wb):
    mesh = plsc.VectorSubcoreMesh(core_axis_name="c", subcore_axis_name="s")
    kern = functools.partial(
        pl.kernel,
        mesh=mesh,
        compiler_params=pltpu.CompilerParams(
            needs_layout_passes=False, disable_bounds_checks=True),
        out_type=jax.ShapeDtypeStruct((BATCH,), jnp.float32),
        scratch_types=[
            pltpu.VMEM((B_PER_W,), jnp.int32),
            pltpu.VMEM((B_PER_W // 8, 8, EMB_DIM), jnp.float32),
            pltpu.VMEM((80,), jnp.float32),
            pltpu.VMEM((B_PER_W,), jnp.float32),
            pltpu.SemaphoreType.DMA,
        ],
    )(_ncf_body)
    return kern(user_ids, item_ids, user_table, item_table, wb)


def kernel(user_ids, item_ids, user_table, item_table, W, b):
    wb = jnp.concatenate(
        [jnp.zeros((1,), jnp.float32), W.reshape(2 * EMB_DIM), b,
         jnp.zeros((14,), jnp.float32)])
    out = _ncf(user_ids.astype(jnp.int32), item_ids.astype(jnp.int32),
               user_table, item_table, wb)
    return out.reshape(BATCH, 1)


# speedup vs baseline: 1.5410x; 1.0001x over previous
import functools

import jax
import jax.numpy as jnp
from jax import lax
from jax.experimental import pallas as pl
from jax.experimental.pallas import tpu as pltpu
from jax.experimental.pallas import tpu_sc as plsc

BATCH = 16384
EMB_DIM = 32
NUM_WORKERS = 32
B_PER_W = BATCH // NUM_WORKERS
GROUPS = B_PER_W // 16
N_USERS = 1000000
N_ITEMS = 100000


def _ncf_body(uid_hbm, iid_hbm, utab_hbm, itab_hbm, wb_hbm, out_hbm,
              idx_v, rows_v, wb_v, out_v, sem):
    cid = lax.axis_index("c")
    sid = lax.axis_index("s")
    wid = sid * 2 + cid
    base = wid * B_PER_W

    pltpu.sync_copy(wb_hbm, wb_v)
    lane = lax.iota(jnp.int32, 16)
    bias = plsc.load_gather(
        wb_v, [jnp.full((16,), 2 * EMB_DIM + 1, jnp.int32)])

    utab3 = utab_hbm.reshape(N_USERS // 8, 8, EMB_DIM)
    itab3 = itab_hbm.reshape(N_ITEMS // 8, 8, EMB_DIM)

    def run_pass(id_hbm, tab3, w_base, first):
        pltpu.sync_copy(id_hbm.at[pl.ds(base, B_PER_W)], idx_v)

        def dma_body(k, _):
            v = idx_v[pl.ds(k * 16, 16)]
            slab = lax.shift_right_logical(v, 3)
            for l in range(16):
                pltpu.async_copy(
                    tab3.at[slab[l], pl.ds(v[l] & 7, 1)],
                    rows_v.at[2 * k + l // 8, pl.ds(l % 8, 1)], sem)
            return 0

        lax.fori_loop(0, GROUPS, dma_body, 0)
        pltpu.make_async_copy(
            tab3.at[pl.ds(0, B_PER_W // 8)], rows_v, sem).wait()

        def group_body(g, _):
            n = g * 16 + lane
            slab = lax.shift_right_logical(n, 3)
            r = n & 7
            if first:
                acc = bias
            else:
                acc = out_v[pl.ds(g * 16, 16)]
            for j in range(EMB_DIM):
                w = plsc.load_gather(
                    wb_v, [jnp.full((16,), w_base + j + 1, jnp.int32)])
                vals = plsc.load_gather(
                    rows_v, [slab, r, jnp.full((16,), j, jnp.int32)])
                acc = acc + vals * w
            out_v[pl.ds(g * 16, 16)] = acc
            return 0

        lax.fori_loop(0, GROUPS, group_body, 0)

    run_pass(uid_hbm, utab3, 0, True)
    run_pass(iid_hbm, itab3, EMB_DIM, False)

    pltpu.sync_copy(out_v, out_hbm.at[pl.ds(base, B_PER_W)])


@jax.jit
def _ncf(user_ids, item_ids, user_table, item_table, wb):
    mesh = plsc.VectorSubcoreMesh(core_axis_name="c", subcore_axis_name="s")
    kern = functools.partial(
        pl.kernel,
        mesh=mesh,
        compiler_params=pltpu.CompilerParams(
            needs_layout_passes=False, disable_bounds_checks=True,
            skip_device_barrier=True),
        out_type=jax.ShapeDtypeStruct((BATCH,), jnp.float32),
        scratch_types=[
            pltpu.VMEM((B_PER_W,), jnp.int32),
            pltpu.VMEM((B_PER_W // 8, 8, EMB_DIM), jnp.float32),
            pltpu.VMEM((80,), jnp.float32),
            pltpu.VMEM((B_PER_W,), jnp.float32),
            pltpu.SemaphoreType.DMA,
        ],
    )(_ncf_body)
    return kern(user_ids, item_ids, user_table, item_table, wb)


def kernel(user_ids, item_ids, user_table, item_table, W, b):
    wb = jnp.concatenate(
        [jnp.zeros((1,), jnp.float32), W.reshape(2 * EMB_DIM), b,
         jnp.zeros((14,), jnp.float32)])
    out = _ncf(user_ids.astype(jnp.int32), item_ids.astype(jnp.int32),
               user_table, item_table, wb)
    return out.reshape(BATCH, 1)
